# baseline (speedup 1.0000x reference)
import functools

import jax
import jax.numpy as jnp
from jax.experimental import pallas as pl
from jax.experimental.pallas import tpu as pltpu

K = 5
LAMBDA_REP = 1.0
DELTA = 0.01
S = 2.0


def _repulsion_kernel(x_ref, out_ref, *, inv_scale):
    b = pl.program_id(0)

    @pl.when(b == 0)
    def _init():
        out_ref[...] = jnp.zeros_like(out_ref)

    step_total = jnp.zeros((), dtype=jnp.float32)
    for i in range(x_ref.shape[0]):
        step_total = step_total + _one_batch(x_ref[i])
    out_ref[...] += (step_total * inv_scale).reshape(1, 1)


def _one_batch(x):
    n = x.shape[0]
    sqd = (jnp.sum(x * x, axis=1) + (0.5 * DELTA))[:, None]
    hi = sqd.astype(jnp.bfloat16).astype(jnp.float32)
    lo = sqd - hi
    ones = jnp.ones((n, 1), dtype=jnp.float32)
    a_op = jnp.concatenate([-2.0 * x, ones, ones, hi, lo], axis=1)
    b_op = jnp.concatenate([x, hi, lo, ones, ones], axis=1)
    d2 = jax.lax.dot_general(
        a_op, b_op, (((1,), (1,)), ((), ())),
        preferred_element_type=jnp.float32,
    )
    col = jax.lax.broadcasted_iota(jnp.int32, (1, n), 1)
    row = jax.lax.broadcasted_iota(jnp.int32, (n, 1), 0)
    fac = LAMBDA_REP + (
        jax.lax.broadcasted_iota(jnp.int32, (1, n), 1).astype(jnp.float32)
        - (0.5 * n)
    ) * (LAMBDA_REP * 2.0 ** -22)
    v = jnp.where(row == col, 0.0, fac / d2)

    nsl = n // 128
    assert nsl == 8
    s = [v[:, j * 128:(j + 1) * 128] for j in range(nsl)]

    def ce(a, b):
        return jnp.maximum(a, b), jnp.minimum(a, b)

    p = [ce(s[2 * j], s[2 * j + 1]) for j in range(4)]

    def merge22(a, b):
        c1, mid1 = ce(a[0], b[0])
        mid2, c4 = ce(a[1], b[1])
        c2, c3 = ce(mid1, mid2)
        return c1, c2, c3, c4

    a = merge22(p[0], p[1])
    b = merge22(p[2], p[3])
    o1, omid1 = ce(a[0], b[0])
    omid2 = jnp.maximum(a[2], b[2])
    o2, o3 = ce(omid1, omid2)
    e1, emid1 = ce(a[1], b[1])
    emid2 = jnp.maximum(a[3], b[3])
    e2 = jnp.maximum(emid1, emid2)
    g1 = o1
    g2, g3 = ce(e1, o2)
    g4, g5 = ce(e2, o3)

    h, l2, l3, l4, l5 = g1, g2, g3, g4, g5
    acc = None
    for r in range(K):
        m = jnp.max(h, axis=1, keepdims=True)
        acc = m if acc is None else acc + m
        if r < K - 1:
            mask = h == m
            h = jnp.where(mask, l2, h)
            l2 = jnp.where(mask, l3, l2)
            l3 = jnp.where(mask, l4, l3)
            l4 = jnp.where(mask, l5, l4)
            l5 = jnp.where(mask, 0.0, l5)

    return jnp.sum(acc)


def kernel(pred_poses):
    B, N, D = pred_poses.shape
    k_actual = min(K, N - 1)
    bb = 4
    total = pl.pallas_call(
        functools.partial(
            _repulsion_kernel, inv_scale=1.0 / (B * N * k_actual)),
        grid=(B // bb,),
        in_specs=[pl.BlockSpec((bb, N, D), lambda b: (b, 0, 0))],
        out_specs=pl.BlockSpec((1, 1), lambda b: (0, 0)),
        out_shape=jax.ShapeDtypeStruct((1, 1), jnp.float32),
    )(pred_poses)
    return jnp.reshape(total, ())


# speedup vs baseline: 1.0745x; 1.0745x over previous
import functools

import jax
import jax.numpy as jnp
from jax.experimental import pallas as pl
from jax.experimental.pallas import tpu as pltpu

K = 5
LAMBDA_REP = 1.0
DELTA = 0.01
S = 2.0


def _repulsion_kernel(x_ref, out_ref, *, inv_scale):
    b = pl.program_id(0)

    @pl.when(b == 0)
    def _init():
        out_ref[...] = jnp.zeros_like(out_ref)

    step_total = jnp.zeros((), dtype=jnp.float32)
    for i in range(x_ref.shape[0]):
        step_total = step_total + _one_batch(x_ref[i])
    out_ref[...] += (step_total * inv_scale).reshape(1, 1)


def _one_batch(x):
    n = x.shape[0]
    sqd = (jnp.sum(x * x, axis=1) + (0.5 * DELTA))[:, None]
    hi = sqd.astype(jnp.bfloat16).astype(jnp.float32)
    lo = sqd - hi
    ones = jnp.ones((n, 1), dtype=jnp.float32)
    a_op = jnp.concatenate([-2.0 * x, ones, ones, hi, lo], axis=1)
    b_op = jnp.concatenate([x, hi, lo, ones, ones], axis=1)
    d2 = jax.lax.dot_general(
        a_op, b_op, (((1,), (1,)), ((), ())),
        preferred_element_type=jnp.float32,
    )
    col = jax.lax.broadcasted_iota(jnp.int32, (1, n), 1)
    row = jax.lax.broadcasted_iota(jnp.int32, (n, 1), 0)
    fac = LAMBDA_REP + (
        jax.lax.broadcasted_iota(jnp.int32, (1, n), 1).astype(jnp.float32)
        - (0.5 * n)
    ) * (LAMBDA_REP * 2.0 ** -22)
    v = jnp.where(row == col, 0.0, fac / d2)

    nsl = n // 128
    assert nsl == 8
    s = [v[:, j * 128:(j + 1) * 128] for j in range(nsl)]

    def ce(a, b):
        return jnp.maximum(a, b), jnp.minimum(a, b)

    p = [ce(s[2 * j], s[2 * j + 1]) for j in range(4)]

    def merge22(a, b):
        c1, mid1 = ce(a[0], b[0])
        mid2, c4 = ce(a[1], b[1])
        c2, c3 = ce(mid1, mid2)
        return c1, c2, c3, c4

    a = merge22(p[0], p[1])
    b = merge22(p[2], p[3])
    o1, omid1 = ce(a[0], b[0])
    omid2 = jnp.maximum(a[2], b[2])
    o2, o3 = ce(omid1, omid2)
    e1, emid1 = ce(a[1], b[1])
    emid2 = jnp.maximum(a[3], b[3])
    e2 = jnp.maximum(emid1, emid2)
    g1 = o1
    g2, g3 = ce(e1, o2)
    g4, g5 = ce(e2, o3)

    h, l2, l3, l4, l5 = g1, g2, g3, g4, g5
    acc = None
    for r in range(K):
        m = jnp.max(h, axis=1, keepdims=True)
        acc = m if acc is None else acc + m
        if r < K - 1:
            mask = h == m
            h = jnp.where(mask, l2, h)
            l2 = jnp.where(mask, l3, l2)
            l3 = jnp.where(mask, l4, l3)
            l4 = jnp.where(mask, l5, l4)
            l5 = jnp.where(mask, 0.0, l5)

    return jnp.sum(acc)


def kernel(pred_poses):
    B, N, D = pred_poses.shape
    k_actual = min(K, N - 1)
    bb = 2
    total = pl.pallas_call(
        functools.partial(
            _repulsion_kernel, inv_scale=1.0 / (B * N * k_actual)),
        grid=(B // bb,),
        in_specs=[pl.BlockSpec((bb, N, D), lambda b: (b, 0, 0))],
        out_specs=pl.BlockSpec((1, 1), lambda b: (0, 0)),
        out_shape=jax.ShapeDtypeStruct((1, 1), jnp.float32),
    )(pred_poses)
    return jnp.reshape(total, ())
